# Initial kernel scaffold; baseline (speedup 1.0000x reference)
import functools

import jax
import jax.numpy as jnp
from jax import lax
from jax.experimental import pallas as pl
from jax.experimental.pallas import tpu as pltpu
from jax.experimental.pallas import tpu_sc as plsc

N_NODES = 10000
D = 128
NC, NS = 2, 16
NW = NC * NS
N_PAD = 10240
E = 320000
E_PAD = 327680
EPW = E_PAD // NW
CHUNK = 128
NCHUNK = EPW // CHUNK
RPT = N_PAD // NS
MM_BLK = 1000


@functools.cache
def _sc_kernels():
    mesh = plsc.VectorSubcoreMesh(
        core_axis_name="c", subcore_axis_name="s", num_cores=NC, num_subcores=NS
    )

    @functools.partial(
        pl.kernel,
        out_type=jax.ShapeDtypeStruct((NC, N_PAD, 16), jnp.float32),
        mesh=mesh,
        scratch_types=[
            pltpu.VMEM((NCHUNK, CHUNK), jnp.int32),
            pltpu.VMEM((CHUNK, 16), jnp.float32),
            pltpu.VMEM((RPT, 16), jnp.float32),
            pltpu.VMEM_SHARED((N_PAD, 16), jnp.float32),
        ],
    )
    def deg_kernel(col_hbm, ones_hbm, zeros_hbm, out_hbm, col_v, ones_v, bounce_v, cnt_sh):
        c = lax.axis_index("c")
        s = lax.axis_index("s")
        wid = c * NS + s
        pltpu.sync_copy(ones_hbm, ones_v)
        pltpu.sync_copy(zeros_hbm, bounce_v)
        pltpu.sync_copy(bounce_v, cnt_sh.at[pl.ds(s * RPT, RPT)])
        pltpu.sync_copy(col_hbm.at[wid], col_v)
        plsc.subcore_barrier()

        def body(j, carry):
            pltpu.sync_copy(ones_v, cnt_sh.at[col_v.at[j]], add=True)
            return carry

        lax.fori_loop(0, NCHUNK, body, 0)
        plsc.subcore_barrier()
        pltpu.sync_copy(cnt_sh.at[pl.ds(s * RPT, RPT)], bounce_v)
        pltpu.sync_copy(bounce_v, out_hbm.at[c, pl.ds(s * RPT, RPT)])

    @functools.partial(
        pl.kernel,
        out_type=jax.ShapeDtypeStruct((NC, N_PAD, D), jnp.float32),
        mesh=mesh,
        scratch_types=[
            pltpu.VMEM((NCHUNK, CHUNK), jnp.int32),
            pltpu.VMEM((NCHUNK, CHUNK), jnp.int32),
            pltpu.VMEM((CHUNK, D), jnp.float32),
            pltpu.VMEM_SHARED((N_PAD, D), jnp.float32),
            pltpu.SemaphoreType.DMA,
        ],
    )
    def agg_kernel(hp_hbm, row_hbm, col_hbm, zeros_hbm, out_hbm,
                   row_v, col_v, buf_v, acc_sh, sem):
        c = lax.axis_index("c")
        s = lax.axis_index("s")
        wid = c * NS + s
        pltpu.sync_copy(zeros_hbm, buf_v)
        for k in range(RPT // CHUNK):
            pltpu.sync_copy(buf_v, acc_sh.at[pl.ds(s * RPT + k * CHUNK, CHUNK)])
        pltpu.sync_copy(row_hbm.at[wid], row_v)
        pltpu.sync_copy(col_hbm.at[wid], col_v)
        plsc.subcore_barrier()

        def body(j, carry):
            pltpu.async_copy(hp_hbm.at[row_v.at[j]], buf_v, sem).wait()
            pltpu.sync_copy(buf_v, acc_sh.at[col_v.at[j]], add=True)
            return carry

        lax.fori_loop(0, NCHUNK, body, 0)
        plsc.subcore_barrier()
        for k in range(RPT // CHUNK):
            pltpu.sync_copy(acc_sh.at[pl.ds(s * RPT + k * CHUNK, CHUNK)], buf_v)
            pltpu.sync_copy(buf_v, out_hbm.at[c, pl.ds(s * RPT + k * CHUNK, CHUNK)])

    return deg_kernel, agg_kernel


def _dinv_body(p0_ref, p1_ref, o_ref):
    o_ref[...] = lax.rsqrt(p0_ref[...] + p1_ref[...] + 1.0)


def _mm_scale_body(x_ref, w_ref, dinv_ref, o_ref):
    g = jnp.dot(x_ref[...], w_ref[...], preferred_element_type=jnp.float32,
                precision=lax.Precision.HIGHEST)
    o_ref[...] = g * dinv_ref[...]


def _combine_mm_body(p0_ref, p1_ref, hp_ref, dinv_ref, b_ref, w_ref, o_ref):
    agg = (p0_ref[...] + p1_ref[...] + hp_ref[...]) * dinv_ref[...] + b_ref[...]
    h = jnp.maximum(agg, 0.0)
    g = jnp.dot(h, w_ref[...], preferred_element_type=jnp.float32,
                precision=lax.Precision.HIGHEST)
    o_ref[...] = g * dinv_ref[...]


def _final_body(q0_ref, q1_ref, hp_ref, dinv_ref, b_ref, o_ref):
    o_ref[...] = (q0_ref[...] + q1_ref[...] + hp_ref[...]) * dinv_ref[...] + b_ref[...]


def _row_spec():
    return pl.BlockSpec((MM_BLK, D), lambda i: (i, 0))


def _full_spec(shape):
    return pl.BlockSpec(shape, lambda i: tuple(0 for _ in shape))


def kernel(x, edge_index, W1, b1, W2, b2):
    f32 = jnp.float32
    row = edge_index[0].astype(jnp.int32)
    col = edge_index[1].astype(jnp.int32)
    pad = E_PAD - E
    row3 = jnp.concatenate([row, jnp.zeros((pad,), jnp.int32)]).reshape(NW, NCHUNK, CHUNK)
    col3 = jnp.concatenate([col, jnp.full((pad,), N_NODES, jnp.int32)]).reshape(NW, NCHUNK, CHUNK)
    ones16 = jnp.ones((CHUNK, 16), f32)
    zeros16 = jnp.zeros((RPT, 16), f32)
    zeros128 = jnp.zeros((CHUNK, D), f32)

    deg_kernel, agg_kernel = _sc_kernels()

    degp = deg_kernel(col3, ones16, zeros16)
    dinv_full = pl.pallas_call(
        _dinv_body,
        out_shape=jax.ShapeDtypeStruct((N_PAD, 16), f32),
    )(degp[0], degp[1])
    dinv = dinv_full[:N_NODES, 0:1]

    grid = (N_NODES // MM_BLK,)
    dinv_spec = pl.BlockSpec((MM_BLK, 1), lambda i: (i, 0))
    b_spec = pl.BlockSpec((1, D), lambda i: (0, 0))

    hp1 = pl.pallas_call(
        _mm_scale_body,
        grid=grid,
        in_specs=[_row_spec(), _full_spec((D, D)), dinv_spec],
        out_specs=_row_spec(),
        out_shape=jax.ShapeDtypeStruct((N_NODES, D), f32),
    )(x, W1, dinv)

    aggp1 = agg_kernel(hp1, row3, col3, zeros128)

    hp2 = pl.pallas_call(
        _combine_mm_body,
        grid=grid,
        in_specs=[_row_spec(), _row_spec(), _row_spec(), dinv_spec, b_spec,
                  _full_spec((D, D))],
        out_specs=_row_spec(),
        out_shape=jax.ShapeDtypeStruct((N_NODES, D), f32),
    )(aggp1[0, :N_NODES], aggp1[1, :N_NODES], hp1, dinv, b1.reshape(1, D), W2)

    aggp2 = agg_kernel(hp2, row3, col3, zeros128)

    out = pl.pallas_call(
        _final_body,
        grid=grid,
        in_specs=[_row_spec(), _row_spec(), _row_spec(), dinv_spec, b_spec],
        out_specs=_row_spec(),
        out_shape=jax.ShapeDtypeStruct((N_NODES, D), f32),
    )(aggp2[0, :N_NODES], aggp2[1, :N_NODES], hp2, dinv, b2.reshape(1, D))
    return out


# speedup vs baseline: 7.6876x; 7.6876x over previous
import functools

import jax
import jax.numpy as jnp
from jax import lax
from jax.experimental import pallas as pl
from jax.experimental.pallas import tpu as pltpu
from jax.experimental.pallas import tpu_sc as plsc

N_NODES = 10000
D = 128
NC, NS = 2, 16
NW = NC * NS
N_PAD = 10240
E = 320000
E_PAD = 327680
EPW = E_PAD // NW
CHUNK = 128
NCHUNK = EPW // CHUNK
RPT = N_PAD // NS
MM_BLK = 1000


def _fill_iota(idx_v, base):
    for m in range(CHUNK // 16):
        idx_v[pl.ds(m * 16, 16)] = lax.iota(jnp.int32, 16) + (base + m * 16)


def _fill_from(idx_v, src_v, j):
    for m in range(CHUNK // 16):
        idx_v[pl.ds(m * 16, 16)] = src_v[j, pl.ds(m * 16, 16)]


@functools.cache
def _sc_kernels():
    mesh = plsc.VectorSubcoreMesh(
        core_axis_name="c", subcore_axis_name="s", num_cores=NC, num_subcores=NS
    )

    @functools.partial(
        pl.kernel,
        out_type=jax.ShapeDtypeStruct((NC, N_PAD, D), jnp.float32),
        mesh=mesh,
        scratch_types=[
            pltpu.VMEM((NCHUNK, CHUNK), jnp.int32),
            pltpu.VMEM((CHUNK,), jnp.int32),
            pltpu.VMEM((CHUNK, D), jnp.float32),
            pltpu.VMEM((CHUNK, D), jnp.float32),
            pltpu.VMEM_SHARED((N_PAD, D), jnp.float32),
        ],
    )
    def deg_kernel(col_hbm, ones_hbm, zeros_hbm, out_hbm, col_v, cidx_v, ones_v,
                   zb_v, cnt_sh):
        c = lax.axis_index("c")
        s = lax.axis_index("s")
        wid = c * NS + s
        pltpu.sync_copy(ones_hbm, ones_v)
        pltpu.sync_copy(zeros_hbm, zb_v)
        pltpu.sync_copy(col_hbm.at[wid], col_v)
        for k in range(RPT // CHUNK):
            _fill_iota(cidx_v, s * RPT + k * CHUNK)
            pltpu.sync_copy(zb_v, cnt_sh.at[cidx_v])
        plsc.subcore_barrier()

        def body(j, carry):
            _fill_from(cidx_v, col_v, j)
            pltpu.sync_copy(ones_v, cnt_sh.at[cidx_v], add=True)
            return carry

        lax.fori_loop(0, NCHUNK, body, 0)
        plsc.subcore_barrier()
        for k in range(RPT // CHUNK):
            _fill_iota(cidx_v, s * RPT + k * CHUNK)
            pltpu.sync_copy(cnt_sh.at[cidx_v], zb_v)
            pltpu.sync_copy(zb_v, out_hbm.at[c, pl.ds(s * RPT + k * CHUNK, CHUNK)])

    @functools.partial(
        pl.kernel,
        out_type=jax.ShapeDtypeStruct((NC, N_PAD, D), jnp.float32),
        mesh=mesh,
        scratch_types=[
            pltpu.VMEM((NCHUNK, CHUNK), jnp.int32),
            pltpu.VMEM((NCHUNK, CHUNK), jnp.int32),
            pltpu.VMEM((CHUNK,), jnp.int32),
            pltpu.VMEM((CHUNK,), jnp.int32),
            pltpu.VMEM((CHUNK, D), jnp.float32),
            pltpu.VMEM_SHARED((N_PAD, D), jnp.float32),
            pltpu.SemaphoreType.DMA,
        ],
    )
    def agg_kernel(hp_hbm, row_hbm, col_hbm, zeros_hbm, out_hbm,
                   row_v, col_v, ridx_v, cidx_v, buf_v, acc_sh, sem):
        c = lax.axis_index("c")
        s = lax.axis_index("s")
        wid = c * NS + s
        pltpu.sync_copy(zeros_hbm, buf_v)
        for k in range(RPT // CHUNK):
            _fill_iota(cidx_v, s * RPT + k * CHUNK)
            pltpu.sync_copy(buf_v, acc_sh.at[cidx_v])
        pltpu.sync_copy(row_hbm.at[wid], row_v)
        pltpu.sync_copy(col_hbm.at[wid], col_v)
        plsc.subcore_barrier()

        def body(j, carry):
            _fill_from(ridx_v, row_v, j)
            _fill_from(cidx_v, col_v, j)
            pltpu.async_copy(hp_hbm.at[ridx_v], buf_v, sem).wait()
            pltpu.sync_copy(buf_v, acc_sh.at[cidx_v], add=True)
            return carry

        lax.fori_loop(0, NCHUNK, body, 0)
        plsc.subcore_barrier()
        for k in range(RPT // CHUNK):
            _fill_iota(cidx_v, s * RPT + k * CHUNK)
            pltpu.sync_copy(acc_sh.at[cidx_v], buf_v)
            pltpu.sync_copy(buf_v, out_hbm.at[c, pl.ds(s * RPT + k * CHUNK, CHUNK)])

    return deg_kernel, agg_kernel


def _dinv_body(p0_ref, p1_ref, o_ref):
    o_ref[...] = lax.rsqrt(p0_ref[...] + p1_ref[...] + 1.0)


def _mm_scale_body(x_ref, w_ref, dinv_ref, o_ref):
    g = jnp.dot(x_ref[...], w_ref[...], preferred_element_type=jnp.float32,
                precision=lax.Precision.HIGHEST)
    o_ref[...] = g * dinv_ref[...]


def _combine_mm_body(p0_ref, p1_ref, hp_ref, dinv_ref, b_ref, w_ref, o_ref):
    agg = (p0_ref[...] + p1_ref[...] + hp_ref[...]) * dinv_ref[...] + b_ref[...]
    h = jnp.maximum(agg, 0.0)
    g = jnp.dot(h, w_ref[...], preferred_element_type=jnp.float32,
                precision=lax.Precision.HIGHEST)
    o_ref[...] = g * dinv_ref[...]


def _final_body(q0_ref, q1_ref, hp_ref, dinv_ref, b_ref, o_ref):
    o_ref[...] = (q0_ref[...] + q1_ref[...] + hp_ref[...]) * dinv_ref[...] + b_ref[...]


def _row_spec():
    return pl.BlockSpec((MM_BLK, D), lambda i: (i, 0))


def _full_spec(shape):
    return pl.BlockSpec(shape, lambda i: tuple(0 for _ in shape))


def kernel(x, edge_index, W1, b1, W2, b2):
    f32 = jnp.float32
    row = edge_index[0].astype(jnp.int32)
    col = edge_index[1].astype(jnp.int32)
    pad = E_PAD - E
    row3 = jnp.concatenate([row, jnp.zeros((pad,), jnp.int32)]).reshape(NW, NCHUNK, CHUNK)
    col3 = jnp.concatenate([col, jnp.full((pad,), N_NODES, jnp.int32)]).reshape(NW, NCHUNK, CHUNK)
    ones128 = jnp.ones((CHUNK, D), f32)
    zeros128 = jnp.zeros((CHUNK, D), f32)

    deg_kernel, agg_kernel = _sc_kernels()

    degp = deg_kernel(col3, ones128, zeros128)
    dinv_full = pl.pallas_call(
        _dinv_body,
        out_shape=jax.ShapeDtypeStruct((N_PAD, D), f32),
    )(degp[0], degp[1])
    dinv = dinv_full[:N_NODES, 0:1]

    grid = (N_NODES // MM_BLK,)
    dinv_spec = pl.BlockSpec((MM_BLK, 1), lambda i: (i, 0))
    b_spec = pl.BlockSpec((1, D), lambda i: (0, 0))

    hp1 = pl.pallas_call(
        _mm_scale_body,
        grid=grid,
        in_specs=[_row_spec(), _full_spec((D, D)), dinv_spec],
        out_specs=_row_spec(),
        out_shape=jax.ShapeDtypeStruct((N_NODES, D), f32),
    )(x, W1, dinv)

    aggp1 = agg_kernel(hp1, row3, col3, zeros128)

    hp2 = pl.pallas_call(
        _combine_mm_body,
        grid=grid,
        in_specs=[_row_spec(), _row_spec(), _row_spec(), dinv_spec, b_spec,
                  _full_spec((D, D))],
        out_specs=_row_spec(),
        out_shape=jax.ShapeDtypeStruct((N_NODES, D), f32),
    )(aggp1[0, :N_NODES], aggp1[1, :N_NODES], hp1, dinv, b1.reshape(1, D), W2)

    aggp2 = agg_kernel(hp2, row3, col3, zeros128)

    out = pl.pallas_call(
        _final_body,
        grid=grid,
        in_specs=[_row_spec(), _row_spec(), _row_spec(), dinv_spec, b_spec],
        out_specs=_row_spec(),
        out_shape=jax.ShapeDtypeStruct((N_NODES, D), f32),
    )(aggp2[0, :N_NODES], aggp2[1, :N_NODES], hp2, dinv, b2.reshape(1, D))
    return out


# speedup vs baseline: 8.5319x; 1.1098x over previous
import functools

import jax
import jax.numpy as jnp
from jax import lax
from jax.experimental import pallas as pl
from jax.experimental.pallas import tpu as pltpu
from jax.experimental.pallas import tpu_sc as plsc

N_NODES = 10000
D = 128
NC, NS = 2, 16
NW = NC * NS
N_PAD = 10240
E = 320000
E_PAD = 327680
EPW = E_PAD // NW
CHUNK = 128
NCHUNK = EPW // CHUNK
RPT = N_PAD // NS
MM_BLK = 1000


def _fill_iota(idx_v, base):
    for m in range(CHUNK // 16):
        idx_v[pl.ds(m * 16, 16)] = lax.iota(jnp.int32, 16) + (base + m * 16)


def _fill_from(idx_v, src_v, j):
    for m in range(CHUNK // 16):
        idx_v[pl.ds(m * 16, 16)] = src_v[j, pl.ds(m * 16, 16)]


@functools.cache
def _sc_kernels():
    mesh = plsc.VectorSubcoreMesh(
        core_axis_name="c", subcore_axis_name="s", num_cores=NC, num_subcores=NS
    )

    @functools.partial(
        pl.kernel,
        out_type=jax.ShapeDtypeStruct((NC, N_PAD, D), jnp.float32),
        mesh=mesh,
        scratch_types=[
            pltpu.VMEM((NCHUNK, CHUNK), jnp.int32),
            pltpu.VMEM((CHUNK,), jnp.int32),
            pltpu.VMEM((CHUNK, D), jnp.float32),
            pltpu.VMEM((CHUNK, D), jnp.float32),
            pltpu.VMEM_SHARED((N_PAD, D), jnp.float32),
        ],
    )
    def deg_kernel(col_hbm, ones_hbm, zeros_hbm, out_hbm, col_v, cidx_v, ones_v,
                   zb_v, cnt_sh):
        c = lax.axis_index("c")
        s = lax.axis_index("s")
        wid = c * NS + s
        pltpu.sync_copy(ones_hbm, ones_v)
        pltpu.sync_copy(zeros_hbm, zb_v)
        pltpu.sync_copy(col_hbm.at[wid], col_v)
        for k in range(RPT // CHUNK):
            _fill_iota(cidx_v, s * RPT + k * CHUNK)
            pltpu.sync_copy(zb_v, cnt_sh.at[cidx_v])
        plsc.subcore_barrier()

        def body(j, carry):
            _fill_from(cidx_v, col_v, j)
            pltpu.sync_copy(ones_v, cnt_sh.at[cidx_v], add=True)
            return carry

        lax.fori_loop(0, NCHUNK, body, 0)
        plsc.subcore_barrier()
        for k in range(RPT // CHUNK):
            _fill_iota(cidx_v, s * RPT + k * CHUNK)
            pltpu.sync_copy(cnt_sh.at[cidx_v], zb_v)
            pltpu.sync_copy(zb_v, out_hbm.at[c, pl.ds(s * RPT + k * CHUNK, CHUNK)])

    @functools.partial(
        pl.kernel,
        out_type=jax.ShapeDtypeStruct((NC, N_PAD, D), jnp.float32),
        mesh=mesh,
        scratch_types=[
            pltpu.VMEM((CHUNK,), jnp.int32),
            pltpu.VMEM((CHUNK,), jnp.int32),
            pltpu.VMEM((CHUNK,), jnp.int32),
            pltpu.VMEM((CHUNK, D), jnp.float32),
            pltpu.VMEM((CHUNK, D), jnp.float32),
            pltpu.VMEM_SHARED((N_PAD, D), jnp.float32),
            pltpu.SemaphoreType.DMA,
            pltpu.SemaphoreType.DMA,
        ],
    )
    def agg_kernel(hp_hbm, row_hbm, col_hbm, zeros_hbm, out_hbm,
                   ridxa_v, ridxb_v, cidx_v, bufa_v, bufb_v,
                   acc_sh, sema, semb):
        c = lax.axis_index("c")
        s = lax.axis_index("s")
        wid = c * NS + s
        pltpu.sync_copy(zeros_hbm, bufa_v)
        for k in range(RPT // CHUNK):
            _fill_iota(cidx_v, s * RPT + k * CHUNK)
            pltpu.sync_copy(bufa_v, acc_sh.at[cidx_v])
        plsc.subcore_barrier()

        pltpu.sync_copy(row_hbm.at[wid, 0], ridxa_v)
        pltpu.async_copy(hp_hbm.at[ridxa_v], bufa_v, sema)

        def body(i, carry):
            pltpu.sync_copy(row_hbm.at[wid, 2 * i + 1], ridxb_v)
            cpb = pltpu.async_copy(hp_hbm.at[ridxb_v], bufb_v, semb)
            pltpu.make_async_copy(hp_hbm.at[ridxa_v], bufa_v, sema).wait()
            pltpu.sync_copy(col_hbm.at[wid, 2 * i], cidx_v)
            pltpu.sync_copy(bufa_v, acc_sh.at[cidx_v], add=True)

            @pl.when(i < NCHUNK // 2 - 1)
            def _():
                pltpu.sync_copy(row_hbm.at[wid, 2 * i + 2], ridxa_v)
                pltpu.async_copy(hp_hbm.at[ridxa_v], bufa_v, sema)

            cpb.wait()
            pltpu.sync_copy(col_hbm.at[wid, 2 * i + 1], cidx_v)
            pltpu.sync_copy(bufb_v, acc_sh.at[cidx_v], add=True)
            return carry

        lax.fori_loop(0, NCHUNK // 2, body, 0)
        plsc.subcore_barrier()
        for k in range(RPT // CHUNK):
            _fill_iota(cidx_v, s * RPT + k * CHUNK)
            pltpu.sync_copy(acc_sh.at[cidx_v], bufa_v)
            pltpu.sync_copy(bufa_v, out_hbm.at[c, pl.ds(s * RPT + k * CHUNK, CHUNK)])

    return deg_kernel, agg_kernel


def _dinv_body(p0_ref, p1_ref, o_ref):
    o_ref[...] = lax.rsqrt(p0_ref[...] + p1_ref[...] + 1.0)


def _mm_scale_body(x_ref, w_ref, dinv_ref, o_ref):
    g = jnp.dot(x_ref[...], w_ref[...], preferred_element_type=jnp.float32,
                precision=lax.Precision.HIGHEST)
    o_ref[...] = g * dinv_ref[...]


def _combine_mm_body(p0_ref, p1_ref, hp_ref, dinv_ref, b_ref, w_ref, o_ref):
    agg = (p0_ref[...] + p1_ref[...] + hp_ref[...]) * dinv_ref[...] + b_ref[...]
    h = jnp.maximum(agg, 0.0)
    g = jnp.dot(h, w_ref[...], preferred_element_type=jnp.float32,
                precision=lax.Precision.HIGHEST)
    o_ref[...] = g * dinv_ref[...]


def _final_body(q0_ref, q1_ref, hp_ref, dinv_ref, b_ref, o_ref):
    o_ref[...] = (q0_ref[...] + q1_ref[...] + hp_ref[...]) * dinv_ref[...] + b_ref[...]


def _row_spec():
    return pl.BlockSpec((MM_BLK, D), lambda i: (i, 0))


def _full_spec(shape):
    return pl.BlockSpec(shape, lambda i: tuple(0 for _ in shape))


def kernel(x, edge_index, W1, b1, W2, b2):
    f32 = jnp.float32
    row = edge_index[0].astype(jnp.int32)
    col = edge_index[1].astype(jnp.int32)
    pad = E_PAD - E
    row3 = jnp.concatenate([row, jnp.zeros((pad,), jnp.int32)]).reshape(NW, NCHUNK, CHUNK)
    col3 = jnp.concatenate([col, jnp.full((pad,), N_NODES, jnp.int32)]).reshape(NW, NCHUNK, CHUNK)
    ones128 = jnp.ones((CHUNK, D), f32)
    zeros128 = jnp.zeros((CHUNK, D), f32)

    deg_kernel, agg_kernel = _sc_kernels()

    degp = deg_kernel(col3, ones128, zeros128)
    dinv_full = pl.pallas_call(
        _dinv_body,
        out_shape=jax.ShapeDtypeStruct((N_PAD, D), f32),
    )(degp[0], degp[1])
    dinv = dinv_full[:N_NODES, 0:1]

    grid = (N_NODES // MM_BLK,)
    dinv_spec = pl.BlockSpec((MM_BLK, 1), lambda i: (i, 0))
    b_spec = pl.BlockSpec((1, D), lambda i: (0, 0))

    hp1 = pl.pallas_call(
        _mm_scale_body,
        grid=grid,
        in_specs=[_row_spec(), _full_spec((D, D)), dinv_spec],
        out_specs=_row_spec(),
        out_shape=jax.ShapeDtypeStruct((N_NODES, D), f32),
    )(x, W1, dinv)

    aggp1 = agg_kernel(hp1, row3, col3, zeros128)

    hp2 = pl.pallas_call(
        _combine_mm_body,
        grid=grid,
        in_specs=[_row_spec(), _row_spec(), _row_spec(), dinv_spec, b_spec,
                  _full_spec((D, D))],
        out_specs=_row_spec(),
        out_shape=jax.ShapeDtypeStruct((N_NODES, D), f32),
    )(aggp1[0, :N_NODES], aggp1[1, :N_NODES], hp1, dinv, b1.reshape(1, D), W2)

    aggp2 = agg_kernel(hp2, row3, col3, zeros128)

    out = pl.pallas_call(
        _final_body,
        grid=grid,
        in_specs=[_row_spec(), _row_spec(), _row_spec(), dinv_spec, b_spec],
        out_specs=_row_spec(),
        out_shape=jax.ShapeDtypeStruct((N_NODES, D), f32),
    )(aggp2[0, :N_NODES], aggp2[1, :N_NODES], hp2, dinv, b2.reshape(1, D))
    return out
